# baseline (speedup 1.0000x reference)
import functools

import jax
import jax.numpy as jnp
from jax import lax
from jax.experimental import pallas as pl
from jax.experimental.pallas import tpu as pltpu
from jax.experimental.pallas import tpu_sc as plsc

_VOC = 1_000_000
_DIM = 16
_N = 3_276_800

_NC = 2
_NS = 16
_NW = _NC * _NS
_VPAD = 1_048_576

_MESH = plsc.VectorSubcoreMesh(
    core_axis_name="c", subcore_axis_name="s", num_cores=_NC, num_subcores=_NS
)

_PER_W = _N // _NW
_SEG = 128
_HK = 16
_NBLK = 5
_BLK_ROWS = _PER_W // (_NBLK * _SEG)
_HGRP = _BLK_ROWS // _HK
_ZCH = 8192
_ZPT = _VPAD // _NS


def _hist_body(x_ref, cnt_ref, idxa, idxb, ones_v, zero_v, cnt_sp, sem, zsem,
               ssem):
    cid = lax.axis_index("c")
    sid = lax.axis_index("s")
    wid = cid * _NS + sid

    def obody(i, _):
        ones_v[pl.ds(i * 16, 16)] = jnp.ones((16,), jnp.float32)
        return 0

    lax.fori_loop(0, _SEG // 16, obody, 0)

    def zbody(i, _):
        zero_v[pl.ds(i * 16, 16)] = jnp.zeros((16,), jnp.float32)
        return 0

    lax.fori_loop(0, _ZCH // 16, zbody, 0)

    stage = pltpu.async_copy(x_ref.at[wid, 0], idxa, ssem)
    zcopies = [
        pltpu.async_copy(
            zero_v, cnt_sp.at[pl.ds(sid * _ZPT + i * _ZCH, _ZCH)], zsem
        )
        for i in range(_ZPT // _ZCH)
    ]
    for c in zcopies:
        c.wait()
    plsc.subcore_barrier()

    def fire(idx_v, g):
        for j in range(_HK):
            pltpu.async_copy(
                ones_v, cnt_sp.at[idx_v.at[g * _HK + j]], sem, add=True
            )

    def drain():
        pltpu.make_async_copy(
            cnt_ref.at[0, pl.ds(0, _HK * _SEG)],
            zero_v.at[pl.ds(0, _HK * _SEG)],
            sem,
        ).wait()

    bufs = (idxa, idxb)
    for blk in range(_NBLK):
        idx_v = bufs[blk % 2]
        stage.wait()
        if blk + 1 < _NBLK:
            stage = pltpu.async_copy(
                x_ref.at[wid, blk + 1], bufs[(blk + 1) % 2], ssem
            )
        fire(idx_v, 0)
        fire(idx_v, 1)

        def step(g, _):
            fire(idx_v, g + 2)
            drain()
            return 0

        lax.fori_loop(0, _HGRP - 2, step, 0)
        drain()
        drain()

    plsc.subcore_barrier()
    pltpu.sync_copy(
        cnt_sp.at[pl.ds(sid * _ZPT, _ZPT)],
        cnt_ref.at[cid, pl.ds(sid * _ZPT, _ZPT)],
    )


_sc_hist = functools.partial(
    pl.kernel,
    out_type=jax.ShapeDtypeStruct((_NC, _VPAD), jnp.float32),
    mesh=_MESH,
    scratch_types=[
        pltpu.VMEM((_BLK_ROWS, _SEG), jnp.int32),
        pltpu.VMEM((_BLK_ROWS, _SEG), jnp.int32),
        pltpu.VMEM((_SEG,), jnp.float32),
        pltpu.VMEM((_ZCH,), jnp.float32),
        pltpu.VMEM_SHARED((_VPAD,), jnp.float32),
        pltpu.SemaphoreType.DMA,
        pltpu.SemaphoreType.DMA,
        pltpu.SemaphoreType.DMA,
    ],
)(_hist_body)


_TL = 65_536
_TG = _VPAD // _TL


def _tc_wsum_body(c_ref, w_ref, o_ref):
    i = pl.program_id(0)

    @pl.when(i == 0)
    def _init():
        o_ref[...] = jnp.zeros_like(o_ref)

    c = c_ref[0:1, :] + c_ref[1:2, :]
    wb = w_ref[...]
    v = i * _TL + jax.lax.broadcasted_iota(jnp.int32, (_DIM, _TL), 1)
    prod = jnp.where(v < _VOC, wb * c, jnp.float32(0.0))
    o_ref[...] += jnp.sum(prod, axis=1, keepdims=True)


_tc_wsum = pl.pallas_call(
    _tc_wsum_body,
    grid=(_TG,),
    in_specs=[
        pl.BlockSpec((_NC, _TL), lambda i: (0, i)),
        pl.BlockSpec((_DIM, _TL), lambda i: (0, i)),
    ],
    out_specs=pl.BlockSpec((_DIM, 1), lambda i: (0, 0)),
    out_shape=jax.ShapeDtypeStruct((_DIM, 1), jnp.float32),
)


def kernel(x, weight):
    x4 = x.astype(jnp.int32).reshape(_NW, _NBLK, _BLK_ROWS, _SEG)
    counts = _sc_hist(x4)
    o = _tc_wsum(counts, weight.T)
    return o.reshape(1, _DIM) * jnp.float32(1.0 / _N)


# speedup vs baseline: 1.0277x; 1.0277x over previous
import functools

import jax
import jax.numpy as jnp
from jax import lax
from jax.experimental import pallas as pl
from jax.experimental.pallas import tpu as pltpu
from jax.experimental.pallas import tpu_sc as plsc

_VOC = 1_000_000
_DIM = 16
_N = 3_276_800

_NC = 2
_NS = 16
_NW = _NC * _NS
_VPAD = 1_048_576

_MESH = plsc.VectorSubcoreMesh(
    core_axis_name="c", subcore_axis_name="s", num_cores=_NC, num_subcores=_NS
)

_PER_W = _N // _NW
_SEG = 128
_HK = 8
_NBLK = 5
_BLK_ROWS = _PER_W // (_NBLK * _SEG)
_HGRP = _BLK_ROWS // _HK
_ZCH = 8192
_ZPT = _VPAD // _NS


def _hist_body(x_ref, cnt_ref, idxa, idxb, ones_v, zero_v, cnt_sp, sem, zsem,
               ssem):
    cid = lax.axis_index("c")
    sid = lax.axis_index("s")
    wid = cid * _NS + sid

    def obody(i, _):
        ones_v[pl.ds(i * 16, 16)] = jnp.ones((16,), jnp.float32)
        return 0

    lax.fori_loop(0, _SEG // 16, obody, 0)

    def zbody(i, _):
        zero_v[pl.ds(i * 16, 16)] = jnp.zeros((16,), jnp.float32)
        return 0

    lax.fori_loop(0, _ZCH // 16, zbody, 0)

    stage = pltpu.async_copy(x_ref.at[wid, 0], idxa, ssem)
    zcopies = [
        pltpu.async_copy(
            zero_v, cnt_sp.at[pl.ds(sid * _ZPT + i * _ZCH, _ZCH)], zsem
        )
        for i in range(_ZPT // _ZCH)
    ]
    for c in zcopies:
        c.wait()
    plsc.subcore_barrier()

    def fire(idx_v, g):
        for j in range(_HK):
            pltpu.async_copy(
                ones_v, cnt_sp.at[idx_v.at[g * _HK + j]], sem, add=True
            )

    def drain():
        pltpu.make_async_copy(
            cnt_ref.at[0, pl.ds(0, _HK * _SEG)],
            zero_v.at[pl.ds(0, _HK * _SEG)],
            sem,
        ).wait()

    bufs = (idxa, idxb)
    for blk in range(_NBLK):
        idx_v = bufs[blk % 2]
        stage.wait()
        if blk + 1 < _NBLK:
            stage = pltpu.async_copy(
                x_ref.at[wid, blk + 1], bufs[(blk + 1) % 2], ssem
            )
        fire(idx_v, 0)
        fire(idx_v, 1)

        def step(g, _):
            fire(idx_v, g + 2)
            drain()
            return 0

        lax.fori_loop(0, _HGRP - 2, step, 0)
        drain()
        drain()

    plsc.subcore_barrier()
    pltpu.sync_copy(
        cnt_sp.at[pl.ds(sid * _ZPT, _ZPT)],
        cnt_ref.at[cid, pl.ds(sid * _ZPT, _ZPT)],
    )


_sc_hist = functools.partial(
    pl.kernel,
    out_type=jax.ShapeDtypeStruct((_NC, _VPAD), jnp.float32),
    mesh=_MESH,
    scratch_types=[
        pltpu.VMEM((_BLK_ROWS, _SEG), jnp.int32),
        pltpu.VMEM((_BLK_ROWS, _SEG), jnp.int32),
        pltpu.VMEM((_SEG,), jnp.float32),
        pltpu.VMEM((_ZCH,), jnp.float32),
        pltpu.VMEM_SHARED((_VPAD,), jnp.float32),
        pltpu.SemaphoreType.DMA,
        pltpu.SemaphoreType.DMA,
        pltpu.SemaphoreType.DMA,
    ],
)(_hist_body)


_TL = 65_536
_TG = _VPAD // _TL


def _tc_wsum_body(c_ref, w_ref, o_ref):
    i = pl.program_id(0)

    @pl.when(i == 0)
    def _init():
        o_ref[...] = jnp.zeros_like(o_ref)

    c = c_ref[0:1, :] + c_ref[1:2, :]
    wb = w_ref[...]
    v = i * _TL + jax.lax.broadcasted_iota(jnp.int32, (_DIM, _TL), 1)
    prod = jnp.where(v < _VOC, wb * c, jnp.float32(0.0))
    o_ref[...] += jnp.sum(prod, axis=1, keepdims=True)


_tc_wsum = pl.pallas_call(
    _tc_wsum_body,
    grid=(_TG,),
    in_specs=[
        pl.BlockSpec((_NC, _TL), lambda i: (0, i)),
        pl.BlockSpec((_DIM, _TL), lambda i: (0, i)),
    ],
    out_specs=pl.BlockSpec((_DIM, 1), lambda i: (0, 0)),
    out_shape=jax.ShapeDtypeStruct((_DIM, 1), jnp.float32),
)


def kernel(x, weight):
    x4 = x.astype(jnp.int32).reshape(_NW, _NBLK, _BLK_ROWS, _SEG)
    counts = _sc_hist(x4)
    o = _tc_wsum(counts, weight.T)
    return o.reshape(1, _DIM) * jnp.float32(1.0 / _N)


# speedup vs baseline: 1.0547x; 1.0262x over previous
import functools

import jax
import jax.numpy as jnp
from jax import lax
from jax.experimental import pallas as pl
from jax.experimental.pallas import tpu as pltpu
from jax.experimental.pallas import tpu_sc as plsc

_VOC = 1_000_000
_DIM = 16
_N = 3_276_800

_NC = 2
_NS = 16
_NW = _NC * _NS
_VPAD = 1_048_576

_MESH = plsc.VectorSubcoreMesh(
    core_axis_name="c", subcore_axis_name="s", num_cores=_NC, num_subcores=_NS
)

_PER_W = _N // _NW
_SEG = 128
_HK = 8
_NBLK = 5
_BLK_ROWS = _PER_W // (_NBLK * _SEG)
_HGRP = _BLK_ROWS // _HK
_ZCH = 8192
_ZPT = _VPAD // _NS


def _hist_body(x_ref, cnt_ref, idxa, idxb, ones_v, zero_v, cnt_sp, sem, zsem,
               ssem):
    cid = lax.axis_index("c")
    sid = lax.axis_index("s")
    wid = cid * _NS + sid

    def obody(i, _):
        ones_v[pl.ds(i * 16, 16)] = jnp.ones((16,), jnp.float32)
        return 0

    lax.fori_loop(0, _SEG // 16, obody, 0)

    def zbody(i, _):
        zero_v[pl.ds(i * 16, 16)] = jnp.zeros((16,), jnp.float32)
        return 0

    lax.fori_loop(0, _ZCH // 16, zbody, 0)

    stage = pltpu.async_copy(x_ref.at[wid, 0], idxa, ssem)
    zcopies = [
        pltpu.async_copy(
            zero_v, cnt_sp.at[pl.ds(sid * _ZPT + i * _ZCH, _ZCH)], zsem
        )
        for i in range(_ZPT // _ZCH)
    ]
    for c in zcopies:
        c.wait()
    plsc.subcore_barrier()

    def fire(idx_v, g):
        for j in range(_HK):
            pltpu.async_copy(
                ones_v, cnt_sp.at[idx_v.at[g * _HK + j]], sem, add=True
            )

    def drain():
        pltpu.make_async_copy(
            cnt_ref.at[0, pl.ds(0, _HK * _SEG)],
            zero_v.at[pl.ds(0, _HK * _SEG)],
            sem,
        ).wait()

    bufs = (idxa, idxb)
    for blk in range(_NBLK):
        idx_v = bufs[blk % 2]
        stage.wait()
        if blk + 1 < _NBLK:
            stage = pltpu.async_copy(
                x_ref.at[wid, blk + 1], bufs[(blk + 1) % 2], ssem
            )
        fire(idx_v, 0)
        fire(idx_v, 1)

        def step(g, _):
            fire(idx_v, g + 2)
            drain()
            return 0

        lax.fori_loop(0, _HGRP - 2, step, 0)
        drain()
        drain()

    plsc.subcore_barrier()
    pltpu.sync_copy(
        cnt_sp.at[pl.ds(sid * _ZPT, _ZPT)],
        cnt_ref.at[cid, pl.ds(sid * _ZPT, _ZPT)],
    )


_sc_hist = functools.partial(
    pl.kernel,
    out_type=jax.ShapeDtypeStruct((_NC, _VPAD), jnp.float32),
    mesh=_MESH,
    scratch_types=[
        pltpu.VMEM((_BLK_ROWS, _SEG), jnp.int32),
        pltpu.VMEM((_BLK_ROWS, _SEG), jnp.int32),
        pltpu.VMEM((_SEG,), jnp.float32),
        pltpu.VMEM((_ZCH,), jnp.float32),
        pltpu.VMEM_SHARED((_VPAD,), jnp.float32),
        pltpu.SemaphoreType.DMA,
        pltpu.SemaphoreType.DMA,
        pltpu.SemaphoreType.DMA,
    ],
)(_hist_body)


_TL = 131_072
_TG = _VPAD // _TL


def _tc_wsum_body(c_ref, w_ref, o_ref):
    i = pl.program_id(0)

    @pl.when(i == 0)
    def _init():
        o_ref[...] = jnp.zeros_like(o_ref)

    c = c_ref[0:1, :] + c_ref[1:2, :]
    wb = w_ref[...]
    v = i * _TL + jax.lax.broadcasted_iota(jnp.int32, (_DIM, _TL), 1)
    prod = jnp.where(v < _VOC, wb * c, jnp.float32(0.0))
    o_ref[...] += jnp.sum(prod, axis=1, keepdims=True)


_tc_wsum = pl.pallas_call(
    _tc_wsum_body,
    grid=(_TG,),
    in_specs=[
        pl.BlockSpec((_NC, _TL), lambda i: (0, i)),
        pl.BlockSpec((_DIM, _TL), lambda i: (0, i)),
    ],
    out_specs=pl.BlockSpec((_DIM, 1), lambda i: (0, 0)),
    out_shape=jax.ShapeDtypeStruct((_DIM, 1), jnp.float32),
)


def kernel(x, weight):
    x4 = x.astype(jnp.int32).reshape(_NW, _NBLK, _BLK_ROWS, _SEG)
    counts = _sc_hist(x4)
    o = _tc_wsum(counts, weight.T)
    return o.reshape(1, _DIM) * jnp.float32(1.0 / _N)


# speedup vs baseline: 1.0552x; 1.0005x over previous
import functools

import jax
import jax.numpy as jnp
from jax import lax
from jax.experimental import pallas as pl
from jax.experimental.pallas import tpu as pltpu
from jax.experimental.pallas import tpu_sc as plsc

_VOC = 1_000_000
_DIM = 16
_N = 3_276_800

_NC = 2
_NS = 16
_NW = _NC * _NS
_VPAD = 1_048_576

_MESH = plsc.VectorSubcoreMesh(
    core_axis_name="c", subcore_axis_name="s", num_cores=_NC, num_subcores=_NS
)

_PER_W = _N // _NW
_SEG = 128
_HK = 8
_NBLK = 5
_BLK_ROWS = _PER_W // (_NBLK * _SEG)
_HGRP = _BLK_ROWS // _HK
_ZCH = 8192
_ZPT = _VPAD // _NS


def _hist_body(x_ref, cnt_ref, idxa, idxb, ones_v, zero_v, cnt_sp, sem, zsem,
               ssem):
    cid = lax.axis_index("c")
    sid = lax.axis_index("s")
    wid = cid * _NS + sid

    def obody(i, _):
        ones_v[pl.ds(i * 16, 16)] = jnp.ones((16,), jnp.float32)
        return 0

    lax.fori_loop(0, _SEG // 16, obody, 0)

    def zbody(i, _):
        zero_v[pl.ds(i * 16, 16)] = jnp.zeros((16,), jnp.float32)
        return 0

    lax.fori_loop(0, _ZCH // 16, zbody, 0)

    stage = pltpu.async_copy(x_ref.at[wid, 0], idxa, ssem)
    zcopies = [
        pltpu.async_copy(
            zero_v, cnt_sp.at[pl.ds(sid * _ZPT + i * _ZCH, _ZCH)], zsem
        )
        for i in range(_ZPT // _ZCH)
    ]
    for c in zcopies:
        c.wait()
    plsc.subcore_barrier()

    def fire(idx_v, g):
        for j in range(_HK):
            pltpu.async_copy(
                ones_v, cnt_sp.at[idx_v.at[g * _HK + j]], sem, add=True
            )

    def drain():
        pltpu.make_async_copy(
            cnt_ref.at[0, pl.ds(0, _HK * _SEG)],
            zero_v.at[pl.ds(0, _HK * _SEG)],
            sem,
        ).wait()

    bufs = (idxa, idxb)
    for blk in range(_NBLK):
        idx_v = bufs[blk % 2]
        stage.wait()
        if blk + 1 < _NBLK:
            stage = pltpu.async_copy(
                x_ref.at[wid, blk + 1], bufs[(blk + 1) % 2], ssem
            )
        fire(idx_v, 0)
        fire(idx_v, 1)

        def step(g, _):
            fire(idx_v, g + 2)
            drain()
            return 0

        lax.fori_loop(0, _HGRP - 2, step, 0)
        drain()
        drain()

    plsc.subcore_barrier()
    pltpu.sync_copy(
        cnt_sp.at[pl.ds(sid * _ZPT, _ZPT)],
        cnt_ref.at[cid, pl.ds(sid * _ZPT, _ZPT)],
    )


_sc_hist = functools.partial(
    pl.kernel,
    out_type=jax.ShapeDtypeStruct((_NC, _VPAD), jnp.float32),
    mesh=_MESH,
    scratch_types=[
        pltpu.VMEM((_BLK_ROWS, _SEG), jnp.int32),
        pltpu.VMEM((_BLK_ROWS, _SEG), jnp.int32),
        pltpu.VMEM((_SEG,), jnp.float32),
        pltpu.VMEM((_ZCH,), jnp.float32),
        pltpu.VMEM_SHARED((_VPAD,), jnp.float32),
        pltpu.SemaphoreType.DMA,
        pltpu.SemaphoreType.DMA,
        pltpu.SemaphoreType.DMA,
    ],
)(_hist_body)


_TL = 262_144
_TG = _VPAD // _TL


def _tc_wsum_body(c_ref, w_ref, o_ref):
    i = pl.program_id(0)

    @pl.when(i == 0)
    def _init():
        o_ref[...] = jnp.zeros_like(o_ref)

    c = c_ref[0:1, :] + c_ref[1:2, :]
    wb = w_ref[...]
    v = i * _TL + jax.lax.broadcasted_iota(jnp.int32, (_DIM, _TL), 1)
    prod = jnp.where(v < _VOC, wb * c, jnp.float32(0.0))
    o_ref[...] += jnp.sum(prod, axis=1, keepdims=True)


_tc_wsum = pl.pallas_call(
    _tc_wsum_body,
    grid=(_TG,),
    in_specs=[
        pl.BlockSpec((_NC, _TL), lambda i: (0, i)),
        pl.BlockSpec((_DIM, _TL), lambda i: (0, i)),
    ],
    out_specs=pl.BlockSpec((_DIM, 1), lambda i: (0, 0)),
    out_shape=jax.ShapeDtypeStruct((_DIM, 1), jnp.float32),
)


def kernel(x, weight):
    x4 = x.astype(jnp.int32).reshape(_NW, _NBLK, _BLK_ROWS, _SEG)
    counts = _sc_hist(x4)
    o = _tc_wsum(counts, weight.T)
    return o.reshape(1, _DIM) * jnp.float32(1.0 / _N)


# speedup vs baseline: 1.0614x; 1.0059x over previous
import functools

import jax
import jax.numpy as jnp
from jax import lax
from jax.experimental import pallas as pl
from jax.experimental.pallas import tpu as pltpu
from jax.experimental.pallas import tpu_sc as plsc

_VOC = 1_000_000
_DIM = 16
_N = 3_276_800

_NC = 2
_NS = 16
_NW = _NC * _NS
_VPAD = 1_048_576

_MESH = plsc.VectorSubcoreMesh(
    core_axis_name="c", subcore_axis_name="s", num_cores=_NC, num_subcores=_NS
)

_PER_W = _N // _NW
_SEG = 128
_HK = 8
_NBLK = 5
_BLK_ROWS = _PER_W // (_NBLK * _SEG)
_HGRP = _BLK_ROWS // _HK
_ZCH = 8192
_ZPT = _VPAD // _NS


def _hist_body(x_ref, cnt_ref, idxa, idxb, ones_v, zero_v, cnt_sp, sem, zsem,
               ssem):
    cid = lax.axis_index("c")
    sid = lax.axis_index("s")
    wid = cid * _NS + sid

    def obody(i, _):
        ones_v[pl.ds(i * 16, 16)] = jnp.ones((16,), jnp.float32)
        return 0

    lax.fori_loop(0, _SEG // 16, obody, 0)

    def zbody(i, _):
        zero_v[pl.ds(i * 16, 16)] = jnp.zeros((16,), jnp.float32)
        return 0

    lax.fori_loop(0, _ZCH // 16, zbody, 0)

    stage = pltpu.async_copy(x_ref.at[wid, 0], idxa, ssem)
    zcopies = [
        pltpu.async_copy(
            zero_v, cnt_sp.at[pl.ds(sid * _ZPT + i * _ZCH, _ZCH)], zsem
        )
        for i in range(_ZPT // _ZCH)
    ]
    for c in zcopies:
        c.wait()
    plsc.subcore_barrier()

    def fire(idx_v, g):
        for j in range(_HK):
            pltpu.async_copy(
                ones_v, cnt_sp.at[idx_v.at[g * _HK + j]], sem, add=True
            )

    def drain():
        pltpu.make_async_copy(
            cnt_ref.at[0, pl.ds(0, _HK * _SEG)],
            zero_v.at[pl.ds(0, _HK * _SEG)],
            sem,
        ).wait()

    bufs = (idxa, idxb)
    for blk in range(_NBLK):
        idx_v = bufs[blk % 2]
        stage.wait()
        if blk + 1 < _NBLK:
            stage = pltpu.async_copy(
                x_ref.at[wid, blk + 1], bufs[(blk + 1) % 2], ssem
            )
        fire(idx_v, 0)
        fire(idx_v, 1)

        def step(g, _):
            fire(idx_v, g + 2)
            drain()
            return 0

        lax.fori_loop(0, _HGRP - 2, step, 0)
        drain()
        drain()

    plsc.subcore_barrier()
    pltpu.sync_copy(
        cnt_sp.at[pl.ds(sid * _ZPT, _ZPT)],
        cnt_ref.at[cid, pl.ds(sid * _ZPT, _ZPT)],
    )


_sc_hist = functools.partial(
    pl.kernel,
    out_type=jax.ShapeDtypeStruct((_NC, _VPAD), jnp.float32),
    mesh=_MESH,
    scratch_types=[
        pltpu.VMEM((_BLK_ROWS, _SEG), jnp.int32),
        pltpu.VMEM((_BLK_ROWS, _SEG), jnp.int32),
        pltpu.VMEM((_SEG,), jnp.float32),
        pltpu.VMEM((_ZCH,), jnp.float32),
        pltpu.VMEM_SHARED((_VPAD,), jnp.float32),
        pltpu.SemaphoreType.DMA,
        pltpu.SemaphoreType.DMA,
        pltpu.SemaphoreType.DMA,
    ],
)(_hist_body)


_TL = 131_072
_TG = _VPAD // _TL


def _tc_wsum_body(c_ref, w_ref, o_ref):
    i = pl.program_id(0)

    @pl.when(i == 0)
    def _init():
        o_ref[...] = jnp.zeros_like(o_ref)

    c = c_ref[0:1, :] + c_ref[1:2, :]
    wb = w_ref[...]
    v = i * _TL + jax.lax.broadcasted_iota(jnp.int32, (_DIM, _TL), 1)
    prod = jnp.where(v < _VOC, wb * c, jnp.float32(0.0))
    o_ref[...] += jnp.sum(prod, axis=1, keepdims=True)


_tc_wsum = pl.pallas_call(
    _tc_wsum_body,
    grid=(_TG,),
    in_specs=[
        pl.BlockSpec((_NC, _TL), lambda i: (0, i)),
        pl.BlockSpec((_DIM, _TL), lambda i: (0, i)),
    ],
    out_specs=pl.BlockSpec((_DIM, 1), lambda i: (0, 0)),
    out_shape=jax.ShapeDtypeStruct((_DIM, 1), jnp.float32),
)


def kernel(x, weight):
    x4 = x.astype(jnp.int32).reshape(_NW, _NBLK, _BLK_ROWS, _SEG)
    counts = _sc_hist(x4)
    o = _tc_wsum(counts, weight.T)
    return o.reshape(1, _DIM) * jnp.float32(1.0 / _N)


# speedup vs baseline: 1.0643x; 1.0028x over previous
import functools

import jax
import jax.numpy as jnp
from jax import lax
from jax.experimental import pallas as pl
from jax.experimental.pallas import tpu as pltpu
from jax.experimental.pallas import tpu_sc as plsc

_VOC = 1_000_000
_DIM = 16
_N = 3_276_800

_NC = 2
_NS = 16
_NW = _NC * _NS
_VPAD = 1_048_576

_MESH = plsc.VectorSubcoreMesh(
    core_axis_name="c", subcore_axis_name="s", num_cores=_NC, num_subcores=_NS
)

_PER_W = _N // _NW
_SEG = 128
_HK = 8
_NBLK = 4
_BLK_ROWS = _PER_W // (_NBLK * _SEG)
_HGRP = _BLK_ROWS // _HK
_ZCH = 8192
_ZPT = _VPAD // _NS


def _hist_body(x_ref, cnt_ref, idxa, idxb, ones_v, zero_v, cnt_sp, sem, zsem,
               ssem):
    cid = lax.axis_index("c")
    sid = lax.axis_index("s")
    wid = cid * _NS + sid

    def obody(i, _):
        ones_v[pl.ds(i * 16, 16)] = jnp.ones((16,), jnp.float32)
        return 0

    lax.fori_loop(0, _SEG // 16, obody, 0)

    def zbody(i, _):
        zero_v[pl.ds(i * 16, 16)] = jnp.zeros((16,), jnp.float32)
        return 0

    lax.fori_loop(0, _ZCH // 16, zbody, 0)

    stage = pltpu.async_copy(x_ref.at[wid, 0], idxa, ssem)
    zcopies = [
        pltpu.async_copy(
            zero_v, cnt_sp.at[pl.ds(sid * _ZPT + i * _ZCH, _ZCH)], zsem
        )
        for i in range(_ZPT // _ZCH)
    ]
    for c in zcopies:
        c.wait()
    plsc.subcore_barrier()

    def fire(idx_v, g):
        for j in range(_HK):
            pltpu.async_copy(
                ones_v, cnt_sp.at[idx_v.at[g * _HK + j]], sem, add=True
            )

    def drain():
        pltpu.make_async_copy(
            cnt_ref.at[0, pl.ds(0, _HK * _SEG)],
            zero_v.at[pl.ds(0, _HK * _SEG)],
            sem,
        ).wait()

    bufs = (idxa, idxb)
    for blk in range(_NBLK):
        idx_v = bufs[blk % 2]
        stage.wait()
        if blk + 1 < _NBLK:
            stage = pltpu.async_copy(
                x_ref.at[wid, blk + 1], bufs[(blk + 1) % 2], ssem
            )
        fire(idx_v, 0)
        fire(idx_v, 1)

        def step(g, _):
            fire(idx_v, g + 2)
            drain()
            return 0

        lax.fori_loop(0, _HGRP - 2, step, 0)
        drain()
        drain()

    plsc.subcore_barrier()
    pltpu.sync_copy(
        cnt_sp.at[pl.ds(sid * _ZPT, _ZPT)],
        cnt_ref.at[cid, pl.ds(sid * _ZPT, _ZPT)],
    )


_sc_hist = functools.partial(
    pl.kernel,
    out_type=jax.ShapeDtypeStruct((_NC, _VPAD), jnp.float32),
    mesh=_MESH,
    scratch_types=[
        pltpu.VMEM((_BLK_ROWS, _SEG), jnp.int32),
        pltpu.VMEM((_BLK_ROWS, _SEG), jnp.int32),
        pltpu.VMEM((_SEG,), jnp.float32),
        pltpu.VMEM((_ZCH,), jnp.float32),
        pltpu.VMEM_SHARED((_VPAD,), jnp.float32),
        pltpu.SemaphoreType.DMA,
        pltpu.SemaphoreType.DMA,
        pltpu.SemaphoreType.DMA,
    ],
)(_hist_body)


_TL = 131_072
_TG = _VPAD // _TL


def _tc_wsum_body(c_ref, w_ref, o_ref):
    i = pl.program_id(0)

    @pl.when(i == 0)
    def _init():
        o_ref[...] = jnp.zeros_like(o_ref)

    c = c_ref[0:1, :] + c_ref[1:2, :]
    wb = w_ref[...]
    v = i * _TL + jax.lax.broadcasted_iota(jnp.int32, (_DIM, _TL), 1)
    prod = jnp.where(v < _VOC, wb * c, jnp.float32(0.0))
    o_ref[...] += jnp.sum(prod, axis=1, keepdims=True)


_tc_wsum = pl.pallas_call(
    _tc_wsum_body,
    grid=(_TG,),
    in_specs=[
        pl.BlockSpec((_NC, _TL), lambda i: (0, i)),
        pl.BlockSpec((_DIM, _TL), lambda i: (0, i)),
    ],
    out_specs=pl.BlockSpec((_DIM, 1), lambda i: (0, 0)),
    out_shape=jax.ShapeDtypeStruct((_DIM, 1), jnp.float32),
)


def kernel(x, weight):
    x4 = x.astype(jnp.int32).reshape(_NW, _NBLK, _BLK_ROWS, _SEG)
    counts = _sc_hist(x4)
    o = _tc_wsum(counts, weight.T)
    return o.reshape(1, _DIM) * jnp.float32(1.0 / _N)
